# baseline (speedup 1.0000x reference)
import jax
import jax.numpy as jnp
from jax import lax
from jax.experimental import pallas as pl
from jax.experimental.pallas import tpu as pltpu
from jax.experimental.pallas import tpu_sc as plsc

F32 = jnp.float32
_M = 10000
_NPAD = 10240
_NC = 2
_NS = 16
_NW = _NC * _NS
_RPS = _NPAD // _NS
_B = 128


def _make_seg_sum(n_chunks, nb, nb0=None, nb1=None):
    if nb0 is None:
        nb0 = nb1 = nb
    fw = 128
    mesh = plsc.VectorSubcoreMesh(core_axis_name="c", subcore_axis_name="s")
    out_type = [jax.ShapeDtypeStruct((_NC, n_chunks, _NPAD, fw), F32)]
    scratch = [
        pltpu.VMEM((_B,), jnp.int32),
        pltpu.VMEM((_B,), jnp.int32),
        pltpu.VMEM((_B,), jnp.int32),
        pltpu.VMEM((_B,), jnp.int32),
        pltpu.VMEM((_B, 64), jnp.int32),
        pltpu.VMEM((_B, 64), jnp.int32),
        pltpu.VMEM((_B, fw), F32),
        pltpu.VMEM_SHARED((_NPAD, fw), F32),
        pltpu.SemaphoreType.DMA,
    ]

    def body(src_hbm, dst_hbm, zeros_hbm, *rest):
        tables = rest[:n_chunks]
        outs = rest[n_chunks]
        (si0, si1, di0, di1, raw0, raw1, rowsf, acc, gsem) = \
            rest[n_chunks + 1:n_chunks + 10]

        cid = lax.axis_index("c")
        sid = lax.axis_index("s")
        wid = sid * _NC + cid
        my = pl.ds(sid * _RPS, _RPS)
        nb_mine = jnp.where(cid == 0, nb0, nb1)
        npairs = nb_mine // 2
        last = nb_mine - 1

        def unpack_scatter(raw, di):
            def row(r, carry):
                for k in range(4):
                    w = raw[r, pl.ds(16 * k, 16)]
                    a = lax.bitcast_convert_type(w << 16, F32)
                    b = lax.bitcast_convert_type(w & jnp.int32(-65536), F32)
                    rowsf[r, pl.ds(16 * k, 16)] = a
                    rowsf[r, pl.ds(64 + 16 * k, 16)] = b
                return carry

            lax.fori_loop(0, _B, row, 0)
            pltpu.sync_copy(rowsf, acc.at[di], add=True)

        for c in range(n_chunks):
            table = tables[c]
            pltpu.sync_copy(zeros_hbm, acc.at[my])
            plsc.subcore_barrier()

            pltpu.sync_copy(src_hbm.at[wid, 0], si0)
            pltpu.sync_copy(dst_hbm.at[wid, 0], di0)
            pltpu.async_copy(table.at[si0], raw0, gsem)
            pltpu.sync_copy(src_hbm.at[wid, 1], si1)
            pltpu.sync_copy(dst_hbm.at[wid, 1], di1)

            def step(i, carry, table=table):
                pltpu.make_async_copy(table.at[si0], raw0, gsem).wait()
                pltpu.async_copy(table.at[si1], raw1, gsem)
                unpack_scatter(raw0, di0)
                j2 = jnp.minimum(2 * i + 2, last)
                pltpu.sync_copy(src_hbm.at[wid, j2], si0)
                pltpu.sync_copy(dst_hbm.at[wid, j2], di0)
                pltpu.make_async_copy(table.at[si1], raw1, gsem).wait()

                @pl.when(i + 1 < npairs)
                def _():
                    pltpu.async_copy(table.at[si0], raw0, gsem)

                unpack_scatter(raw1, di1)
                j3 = jnp.minimum(2 * i + 3, last)
                pltpu.sync_copy(src_hbm.at[wid, j3], si1)
                pltpu.sync_copy(dst_hbm.at[wid, j3], di1)
                return carry

            lax.fori_loop(0, npairs, step, 0)
            plsc.subcore_barrier()
            pltpu.sync_copy(acc.at[my], outs.at[cid, c, my])

    return pl.kernel(body, out_type=out_type, mesh=mesh,
                     scratch_types=scratch,
                     compiler_params=pltpu.CompilerParams(
                         use_tc_tiling_on_sc=False))


def _make_cnt(nb):
    mesh = plsc.VectorSubcoreMesh(core_axis_name="c", subcore_axis_name="s")
    out_type = [jax.ShapeDtypeStruct((_NC, _NPAD, 128), F32)]
    scratch = [
        pltpu.VMEM((nb, _B), jnp.int32),
        pltpu.VMEM((_B, 128), F32),
        pltpu.VMEM_SHARED((_NPAD, 128), F32),
    ]

    def body(dst_hbm, zeros_hbm, ones_hbm, out, dst_v, ones_v, acc):
        cid = lax.axis_index("c")
        sid = lax.axis_index("s")
        wid = sid * _NC + cid
        my = pl.ds(sid * _RPS, _RPS)

        pltpu.sync_copy(dst_hbm.at[wid], dst_v)
        pltpu.sync_copy(ones_hbm, ones_v)
        pltpu.sync_copy(zeros_hbm, acc.at[my])
        plsc.subcore_barrier()

        def step(j, carry):
            pltpu.sync_copy(ones_v, acc.at[dst_v.at[j]], add=True)
            return carry

        lax.fori_loop(0, nb, step, 0)
        plsc.subcore_barrier()
        pltpu.sync_copy(acc.at[my], out.at[cid, my])

    return pl.kernel(body, out_type=out_type, mesh=mesh,
                     scratch_types=scratch)


_RB = 1024


def _sage_compute(p_ref, cnt_ref, xc_refs, wlt_ref, bl_ref, wrt_ref,
                  f_out, n_p, fw):
    cnt = cnt_ref[0, :, 0:1] + cnt_ref[1, :, 0:1]
    rc = 1.0 / jnp.maximum(cnt, 1.0)
    acc = jnp.broadcast_to(bl_ref[...], (_RB, f_out))
    wlt = wlt_ref[...]
    wrt = wrt_ref[...]
    for c in range(n_p):
        mean_c = (p_ref[0, c].astype(F32) + p_ref[1, c].astype(F32)) * rc
        acc = acc + jnp.dot(mean_c, wlt[c * fw:(c + 1) * fw, :],
                            preferred_element_type=F32)
    for c in range(len(xc_refs)):
        acc = acc + jnp.dot(xc_refs[c][...], wrt[c * 128:(c + 1) * 128, :],
                            preferred_element_type=F32)
    return jnp.maximum(acc, 0.0)


def _sage_specs(c_in, n_p, fw):
    in_specs = [
        pl.BlockSpec((2, n_p, _RB, fw), lambda i: (0, 0, i, 0)),
        pl.BlockSpec((2, _RB, 16), lambda i: (0, i, 0)),
    ]
    in_specs += [pl.BlockSpec((_RB, 128), lambda i: (i, 0))
                 for _ in range(c_in)]
    return in_specs


def _sage_layer(c_in, n_p, fw, f_out):
    c_out = f_out // 128

    def body(p_ref, cnt_ref, *rest):
        xc = rest[:c_in]
        wlt, bl, wrt = rest[c_in:c_in + 3]
        outs = rest[c_in + 3:]
        h = _sage_compute(p_ref, cnt_ref, xc, wlt, bl, wrt, f_out, n_p, fw)
        for co in range(c_out):
            outs[co][...] = h[:, co * 128:(co + 1) * 128]

    grid = (_NPAD // _RB,)
    in_specs = _sage_specs(c_in, n_p, fw) + [
        pl.BlockSpec((c_in * 128, f_out), lambda i: (0, 0)),
        pl.BlockSpec((1, f_out), lambda i: (0, 0)),
        pl.BlockSpec((c_in * 128, f_out), lambda i: (0, 0)),
    ]
    out_specs = [pl.BlockSpec((_RB, 128), lambda i: (i, 0))
                 for _ in range(c_out)]
    out_shape = [jax.ShapeDtypeStruct((_NPAD, 128), F32)
                 for _ in range(c_out)]
    return pl.pallas_call(body, grid=grid, in_specs=in_specs,
                          out_specs=out_specs, out_shape=out_shape)


def _sage_mlp(c_in, n_p, fw, f_mid):
    def body(p_ref, cnt_ref, *rest):
        xc = rest[:c_in]
        wlt, bl, wrt = rest[c_in:c_in + 3]
        w1t, b1, w2t, b2, w3t, b3 = rest[c_in + 3:c_in + 9]
        out = rest[c_in + 9]
        h = _sage_compute(p_ref, cnt_ref, xc, wlt, bl, wrt, f_mid, n_p, fw)
        h = jnp.maximum(jnp.dot(h, w1t[...], preferred_element_type=F32)
                        + b1[...], 0.0)
        h = jnp.maximum(jnp.dot(h, w2t[...], preferred_element_type=F32)
                        + b2[...], 0.0)
        h = jnp.maximum(jnp.dot(h, w3t[...], preferred_element_type=F32)
                        + b3[...], 0.0)
        out[...] = h

    grid = (_NPAD // _RB,)
    in_specs = _sage_specs(c_in, n_p, fw) + [
        pl.BlockSpec((c_in * 128, f_mid), lambda i: (0, 0)),
        pl.BlockSpec((1, f_mid), lambda i: (0, 0)),
        pl.BlockSpec((c_in * 128, f_mid), lambda i: (0, 0)),
        pl.BlockSpec((f_mid, 256), lambda i: (0, 0)),
        pl.BlockSpec((1, 256), lambda i: (0, 0)),
        pl.BlockSpec((256, 128), lambda i: (0, 0)),
        pl.BlockSpec((1, 128), lambda i: (0, 0)),
        pl.BlockSpec((128, 128), lambda i: (0, 0)),
        pl.BlockSpec((1, 128), lambda i: (0, 0)),
    ]
    out_specs = pl.BlockSpec((_RB, 128), lambda i: (i, 0))
    out_shape = jax.ShapeDtypeStruct((_NPAD, 128), F32)
    return pl.pallas_call(body, grid=grid, in_specs=in_specs,
                          out_specs=out_specs, out_shape=out_shape)


def _final_matmul():
    MB, NB = 400, 1024
    grid = (_M // MB, pl.cdiv(_M, NB))

    def body(x_ref, y_ref, o_ref):
        o_ref[...] = lax.dot_general(
            x_ref[...], y_ref[...], (((1,), (1,)), ((), ())),
            preferred_element_type=F32)

    return pl.pallas_call(
        body, grid=grid,
        in_specs=[pl.BlockSpec((MB, 128), lambda i, j: (i, 0)),
                  pl.BlockSpec((NB, 128), lambda i, j: (j, 0))],
        out_specs=pl.BlockSpec((MB, NB), lambda i, j: (i, j)),
        out_shape=jax.ShapeDtypeStruct((_M, _M), F32))


def _prep_edges(edge_index):
    src = edge_index[0].astype(jnp.int32)
    dst = edge_index[1].astype(jnp.int32)
    e = src.shape[0]
    nb = -(-e // (_NW * _B))
    epad = _NW * nb * _B
    src = jnp.concatenate([src, jnp.zeros((epad - e,), jnp.int32)])
    dst = jnp.concatenate([dst, jnp.full((epad - e,), _M, jnp.int32)])
    return src.reshape(_NW, nb, _B), dst.reshape(_NW, nb, _B), nb


def _prep_edges_split(edge_index, b0, b1):
    src = edge_index[0].astype(jnp.int32)
    dst = edge_index[1].astype(jnp.int32)
    e = src.shape[0]
    nbmax = max(b0, b1)
    epad = _NS * (b0 + b1) * _B
    src = jnp.concatenate([src, jnp.zeros((epad - e,), jnp.int32)])
    dst = jnp.concatenate([dst, jnp.full((epad - e,), _M, jnp.int32)])

    def arrange(a, pad_val):
        g0 = a[:_NS * b0 * _B].reshape(_NS, b0, _B)
        g1 = a[_NS * b0 * _B:].reshape(_NS, b1, _B)
        g0 = jnp.pad(g0, ((0, 0), (0, nbmax - b0), (0, 0)),
                     constant_values=pad_val)
        g1 = jnp.pad(g1, ((0, 0), (0, nbmax - b1), (0, 0)),
                     constant_values=pad_val)
        return jnp.stack([g0, g1], axis=1).reshape(_NW, nbmax, _B)

    return arrange(src, 0), arrange(dst, _M), nbmax


def _prep_chunks(x):
    n, f = x.shape
    xp = jnp.pad(x, ((0, _NPAD - n), (0, 0)))
    return [xp[:, c * 128:(c + 1) * 128] for c in range(f // 128)]


def _pack128(t):
    tb = t.astype(jnp.bfloat16)
    return lax.bitcast_convert_type(
        jnp.stack([tb[:, :64], tb[:, 64:]], axis=-1), jnp.int32)


def _row(b):
    return b.reshape(1, -1)


def kernel(x_m, x_d, Wl_x1, bl_x1, Wr_x1, Wl_x2, bl_x2, Wr_x2,
           Wl_y1, bl_y1, Wr_y1, Wl_y2, bl_y2, Wr_y2,
           Wx1, bx1, Wx2, bx2, Wx3, bx3,
           Wy1, by1, Wy2, by2, Wy3, by3,
           mm_edge_index, dd_edge_index):
    srcm, dstm, nbm = _prep_edges(mm_edge_index)
    srcd, dstd, nbd = _prep_edges(dd_edge_index)
    _B0, _B1 = 60, 20
    srcms, dstms, nbms = _prep_edges_split(mm_edge_index, _B0, _B1)
    srcds, dstds, nbds = _prep_edges_split(dd_edge_index, _B0, _B1)
    xc = _prep_chunks(x_m)
    yc = _prep_chunks(x_d)

    zer = jnp.zeros((_RPS, 128), F32)
    on128 = jnp.ones((_B, 128), F32)

    CNTx, = _make_cnt(nbm)(dstm, zer, on128)
    CNTx = CNTx[:, :, :16]
    P1x, = _make_seg_sum(2, nbms, _B0, _B1)(
        srcms, dstms, zer, _pack128(xc[0]), _pack128(xc[1]))
    X1c = _sage_layer(2, 2, 128, 512)(
        P1x, CNTx, xc[0], xc[1], Wl_x1.T, _row(bl_x1), Wr_x1.T)
    P2x, = _make_seg_sum(4, nbms, _B0, _B1)(
        srcms, dstms, zer, *[_pack128(t) for t in X1c])
    xf = _sage_mlp(4, 4, 128, 256)(
        P2x, CNTx, *X1c, Wl_x2.T, _row(bl_x2), Wr_x2.T,
        Wx1.T, _row(bx1), Wx2.T, _row(bx2), Wx3.T, _row(bx3))

    CNTy, = _make_cnt(nbd)(dstd, zer, on128)
    CNTy = CNTy[:, :, :16]
    P1y, = _make_seg_sum(1, nbds, _B0, _B1)(srcds, dstds, zer,
                                            _pack128(yc[0]))
    Y1c = _sage_layer(1, 1, 128, 256)(
        P1y, CNTy, yc[0], Wl_y1.T, _row(bl_y1), Wr_y1.T)
    P2y, = _make_seg_sum(2, nbds, _B0, _B1)(
        srcds, dstds, zer, *[_pack128(t) for t in Y1c])
    yf = _sage_mlp(2, 2, 128, 128)(
        P2y, CNTy, *Y1c, Wl_y2.T, _row(bl_y2), Wr_y2.T,
        Wy1.T, _row(by1), Wy2.T, _row(by2), Wy3.T, _row(by3))

    return _final_matmul()(xf, yf)


# speedup vs baseline: 1.1065x; 1.1065x over previous
import jax
import jax.numpy as jnp
from jax import lax
from jax.experimental import pallas as pl
from jax.experimental.pallas import tpu as pltpu
from jax.experimental.pallas import tpu_sc as plsc

F32 = jnp.float32
_M = 10000
_NPAD = 10240
_NC = 2
_NS = 16
_NW = _NC * _NS
_RPS = _NPAD // _NS
_B = 128


def _make_seg_sum(n_chunks, nb, nb0=None, nb1=None):
    if nb0 is None:
        nb0 = nb1 = nb
    fw = 128
    mesh = plsc.VectorSubcoreMesh(core_axis_name="c", subcore_axis_name="s")
    out_type = [jax.ShapeDtypeStruct((_NC, n_chunks, _NPAD, fw), F32)]
    scratch = [
        pltpu.VMEM((_B,), jnp.int32),
        pltpu.VMEM((_B,), jnp.int32),
        pltpu.VMEM((_B,), jnp.int32),
        pltpu.VMEM((_B,), jnp.int32),
        pltpu.VMEM((_B, fw), F32),
        pltpu.VMEM((_B, fw), F32),
        pltpu.VMEM_SHARED((_NPAD, fw), F32),
        pltpu.SemaphoreType.DMA,
    ]

    def body(src_hbm, dst_hbm, zeros_hbm, *rest):
        tables = rest[:n_chunks]
        outs = rest[n_chunks]
        (si0, si1, di0, di1, raw0, raw1, acc, gsem) = \
            rest[n_chunks + 1:n_chunks + 9]

        cid = lax.axis_index("c")
        sid = lax.axis_index("s")
        wid = sid * _NC + cid
        my = pl.ds(sid * _RPS, _RPS)
        nb_mine = jnp.where(cid == 0, nb0, nb1)
        npairs = nb_mine // 2
        last = nb_mine - 1

        def scatter_add(raw, di):
            pltpu.sync_copy(raw, acc.at[di], add=True)

        for c in range(n_chunks):
            table = tables[c]
            pltpu.sync_copy(zeros_hbm, acc.at[my])
            plsc.subcore_barrier()

            pltpu.sync_copy(src_hbm.at[wid, 0], si0)
            pltpu.sync_copy(dst_hbm.at[wid, 0], di0)
            pltpu.async_copy(table.at[si0], raw0, gsem)
            pltpu.sync_copy(src_hbm.at[wid, 1], si1)
            pltpu.sync_copy(dst_hbm.at[wid, 1], di1)

            def step(i, carry, table=table):
                pltpu.make_async_copy(table.at[si0], raw0, gsem).wait()
                pltpu.async_copy(table.at[si1], raw1, gsem)
                scatter_add(raw0, di0)
                j2 = jnp.minimum(2 * i + 2, last)
                pltpu.sync_copy(src_hbm.at[wid, j2], si0)
                pltpu.sync_copy(dst_hbm.at[wid, j2], di0)
                pltpu.make_async_copy(table.at[si1], raw1, gsem).wait()

                @pl.when(i + 1 < npairs)
                def _():
                    pltpu.async_copy(table.at[si0], raw0, gsem)

                scatter_add(raw1, di1)
                j3 = jnp.minimum(2 * i + 3, last)
                pltpu.sync_copy(src_hbm.at[wid, j3], si1)
                pltpu.sync_copy(dst_hbm.at[wid, j3], di1)
                return carry

            lax.fori_loop(0, npairs, step, 0)
            plsc.subcore_barrier()
            pltpu.sync_copy(acc.at[my], outs.at[cid, c, my])

    return pl.kernel(body, out_type=out_type, mesh=mesh,
                     scratch_types=scratch)


def _make_cnt(nb):
    mesh = plsc.VectorSubcoreMesh(core_axis_name="c", subcore_axis_name="s")
    out_type = [jax.ShapeDtypeStruct((_NC, _NPAD, 128), F32)]
    scratch = [
        pltpu.VMEM((nb, _B), jnp.int32),
        pltpu.VMEM((_B, 128), F32),
        pltpu.VMEM_SHARED((_NPAD, 128), F32),
    ]

    def body(dst_hbm, zeros_hbm, ones_hbm, out, dst_v, ones_v, acc):
        cid = lax.axis_index("c")
        sid = lax.axis_index("s")
        wid = sid * _NC + cid
        my = pl.ds(sid * _RPS, _RPS)

        pltpu.sync_copy(dst_hbm.at[wid], dst_v)
        pltpu.sync_copy(ones_hbm, ones_v)
        pltpu.sync_copy(zeros_hbm, acc.at[my])
        plsc.subcore_barrier()

        def step(j, carry):
            pltpu.sync_copy(ones_v, acc.at[dst_v.at[j]], add=True)
            return carry

        lax.fori_loop(0, nb, step, 0)
        plsc.subcore_barrier()
        pltpu.sync_copy(acc.at[my], out.at[cid, my])

    return pl.kernel(body, out_type=out_type, mesh=mesh,
                     scratch_types=scratch)


_RB = 1024


def _sage_compute(p_ref, cnt_ref, xc_refs, wlt_ref, bl_ref, wrt_ref,
                  f_out, n_p, fw):
    cnt = cnt_ref[0, :, 0:1] + cnt_ref[1, :, 0:1]
    rc = 1.0 / jnp.maximum(cnt, 1.0)
    acc = jnp.broadcast_to(bl_ref[...], (_RB, f_out))
    wlt = wlt_ref[...]
    wrt = wrt_ref[...]
    for c in range(n_p):
        mean_c = (p_ref[0, c].astype(F32) + p_ref[1, c].astype(F32)) * rc
        acc = acc + jnp.dot(mean_c, wlt[c * fw:(c + 1) * fw, :],
                            preferred_element_type=F32)
    for c in range(len(xc_refs)):
        acc = acc + jnp.dot(xc_refs[c][...], wrt[c * 128:(c + 1) * 128, :],
                            preferred_element_type=F32)
    return jnp.maximum(acc, 0.0)


def _sage_specs(c_in, n_p, fw):
    in_specs = [
        pl.BlockSpec((2, n_p, _RB, fw), lambda i: (0, 0, i, 0)),
        pl.BlockSpec((2, _RB, 16), lambda i: (0, i, 0)),
    ]
    in_specs += [pl.BlockSpec((_RB, 128), lambda i: (i, 0))
                 for _ in range(c_in)]
    return in_specs


def _sage_layer(c_in, n_p, fw, f_out):
    c_out = f_out // 128

    def body(p_ref, cnt_ref, *rest):
        xc = rest[:c_in]
        wlt, bl, wrt = rest[c_in:c_in + 3]
        outs = rest[c_in + 3:]
        h = _sage_compute(p_ref, cnt_ref, xc, wlt, bl, wrt, f_out, n_p, fw)
        for co in range(c_out):
            outs[co][...] = h[:, co * 128:(co + 1) * 128]

    grid = (_NPAD // _RB,)
    in_specs = _sage_specs(c_in, n_p, fw) + [
        pl.BlockSpec((c_in * 128, f_out), lambda i: (0, 0)),
        pl.BlockSpec((1, f_out), lambda i: (0, 0)),
        pl.BlockSpec((c_in * 128, f_out), lambda i: (0, 0)),
    ]
    out_specs = [pl.BlockSpec((_RB, 128), lambda i: (i, 0))
                 for _ in range(c_out)]
    out_shape = [jax.ShapeDtypeStruct((_NPAD, 128), F32)
                 for _ in range(c_out)]
    return pl.pallas_call(body, grid=grid, in_specs=in_specs,
                          out_specs=out_specs, out_shape=out_shape)


def _sage_mlp(c_in, n_p, fw, f_mid):
    def body(p_ref, cnt_ref, *rest):
        xc = rest[:c_in]
        wlt, bl, wrt = rest[c_in:c_in + 3]
        w1t, b1, w2t, b2, w3t, b3 = rest[c_in + 3:c_in + 9]
        out = rest[c_in + 9]
        h = _sage_compute(p_ref, cnt_ref, xc, wlt, bl, wrt, f_mid, n_p, fw)
        h = jnp.maximum(jnp.dot(h, w1t[...], preferred_element_type=F32)
                        + b1[...], 0.0)
        h = jnp.maximum(jnp.dot(h, w2t[...], preferred_element_type=F32)
                        + b2[...], 0.0)
        h = jnp.maximum(jnp.dot(h, w3t[...], preferred_element_type=F32)
                        + b3[...], 0.0)
        out[...] = h

    grid = (_NPAD // _RB,)
    in_specs = _sage_specs(c_in, n_p, fw) + [
        pl.BlockSpec((c_in * 128, f_mid), lambda i: (0, 0)),
        pl.BlockSpec((1, f_mid), lambda i: (0, 0)),
        pl.BlockSpec((c_in * 128, f_mid), lambda i: (0, 0)),
        pl.BlockSpec((f_mid, 256), lambda i: (0, 0)),
        pl.BlockSpec((1, 256), lambda i: (0, 0)),
        pl.BlockSpec((256, 128), lambda i: (0, 0)),
        pl.BlockSpec((1, 128), lambda i: (0, 0)),
        pl.BlockSpec((128, 128), lambda i: (0, 0)),
        pl.BlockSpec((1, 128), lambda i: (0, 0)),
    ]
    out_specs = pl.BlockSpec((_RB, 128), lambda i: (i, 0))
    out_shape = jax.ShapeDtypeStruct((_NPAD, 128), F32)
    return pl.pallas_call(body, grid=grid, in_specs=in_specs,
                          out_specs=out_specs, out_shape=out_shape)


def _final_matmul():
    MB, NB = 400, 1024
    grid = (_M // MB, pl.cdiv(_M, NB))

    def body(x_ref, y_ref, o_ref):
        o_ref[...] = lax.dot_general(
            x_ref[...], y_ref[...], (((1,), (1,)), ((), ())),
            preferred_element_type=F32)

    return pl.pallas_call(
        body, grid=grid,
        in_specs=[pl.BlockSpec((MB, 128), lambda i, j: (i, 0)),
                  pl.BlockSpec((NB, 128), lambda i, j: (j, 0))],
        out_specs=pl.BlockSpec((MB, NB), lambda i, j: (i, j)),
        out_shape=jax.ShapeDtypeStruct((_M, _M), F32))


def _prep_edges(edge_index):
    src = edge_index[0].astype(jnp.int32)
    dst = edge_index[1].astype(jnp.int32)
    e = src.shape[0]
    nb = -(-e // (_NW * _B))
    epad = _NW * nb * _B
    src = jnp.concatenate([src, jnp.zeros((epad - e,), jnp.int32)])
    dst = jnp.concatenate([dst, jnp.full((epad - e,), _M, jnp.int32)])
    return src.reshape(_NW, nb, _B), dst.reshape(_NW, nb, _B), nb


def _prep_edges_split(edge_index, b0, b1):
    src = edge_index[0].astype(jnp.int32)
    dst = edge_index[1].astype(jnp.int32)
    e = src.shape[0]
    nbmax = max(b0, b1)
    epad = _NS * (b0 + b1) * _B
    src = jnp.concatenate([src, jnp.zeros((epad - e,), jnp.int32)])
    dst = jnp.concatenate([dst, jnp.full((epad - e,), _M, jnp.int32)])

    def arrange(a, pad_val):
        g0 = a[:_NS * b0 * _B].reshape(_NS, b0, _B)
        g1 = a[_NS * b0 * _B:].reshape(_NS, b1, _B)
        g0 = jnp.pad(g0, ((0, 0), (0, nbmax - b0), (0, 0)),
                     constant_values=pad_val)
        g1 = jnp.pad(g1, ((0, 0), (0, nbmax - b1), (0, 0)),
                     constant_values=pad_val)
        return jnp.stack([g0, g1], axis=1).reshape(_NW, nbmax, _B)

    return arrange(src, 0), arrange(dst, _M), nbmax


def _prep_chunks(x):
    n, f = x.shape
    xp = jnp.pad(x, ((0, _NPAD - n), (0, 0)))
    return [xp[:, c * 128:(c + 1) * 128] for c in range(f // 128)]


def _row(b):
    return b.reshape(1, -1)


def kernel(x_m, x_d, Wl_x1, bl_x1, Wr_x1, Wl_x2, bl_x2, Wr_x2,
           Wl_y1, bl_y1, Wr_y1, Wl_y2, bl_y2, Wr_y2,
           Wx1, bx1, Wx2, bx2, Wx3, bx3,
           Wy1, by1, Wy2, by2, Wy3, by3,
           mm_edge_index, dd_edge_index):
    srcm, dstm, nbm = _prep_edges(mm_edge_index)
    srcd, dstd, nbd = _prep_edges(dd_edge_index)
    _B0, _B1 = 60, 20
    srcms, dstms, nbms = _prep_edges_split(mm_edge_index, _B0, _B1)
    srcds, dstds, nbds = _prep_edges_split(dd_edge_index, _B0, _B1)
    xc = _prep_chunks(x_m)
    yc = _prep_chunks(x_d)

    zer = jnp.zeros((_RPS, 128), F32)
    on128 = jnp.ones((_B, 128), F32)

    CNTx, = _make_cnt(nbm)(dstm, zer, on128)
    CNTx = CNTx[:, :, :16]
    P1x, = _make_seg_sum(2, nbms, _B0, _B1)(srcms, dstms, zer, xc[0], xc[1])
    X1c = _sage_layer(2, 2, 128, 512)(
        P1x, CNTx, xc[0], xc[1], Wl_x1.T, _row(bl_x1), Wr_x1.T)
    P2x, = _make_seg_sum(4, nbms, _B0, _B1)(srcms, dstms, zer, *X1c)
    xf = _sage_mlp(4, 4, 128, 256)(
        P2x, CNTx, *X1c, Wl_x2.T, _row(bl_x2), Wr_x2.T,
        Wx1.T, _row(bx1), Wx2.T, _row(bx2), Wx3.T, _row(bx3))

    CNTy, = _make_cnt(nbd)(dstd, zer, on128)
    CNTy = CNTy[:, :, :16]
    P1y, = _make_seg_sum(1, nbds, _B0, _B1)(srcds, dstds, zer, yc[0])
    Y1c = _sage_layer(1, 1, 128, 256)(
        P1y, CNTy, yc[0], Wl_y1.T, _row(bl_y1), Wr_y1.T)
    P2y, = _make_seg_sum(2, nbds, _B0, _B1)(srcds, dstds, zer, *Y1c)
    yf = _sage_mlp(2, 2, 128, 128)(
        P2y, CNTy, *Y1c, Wl_y2.T, _row(bl_y2), Wr_y2.T,
        Wy1.T, _row(by1), Wy2.T, _row(by2), Wy3.T, _row(by3))

    return _final_matmul()(xf.astype(jnp.bfloat16), yf.astype(jnp.bfloat16))
